# Initial kernel scaffold; baseline (speedup 1.0000x reference)
import jax
import jax.numpy as jnp
from jax import lax
from jax.experimental import pallas as pl
from jax.experimental.pallas import tpu as pltpu
from jax.experimental.pallas import tpu_sc as plsc

NTILE = 16
NCORE = 2
LANES = 128
CH = 8
ZR = 625


def _zero_vmem(buf, nrows, h):
    def body(i, _):
        for o in range(0, h, 16):
            buf[i, pl.ds(o, 16)] = jnp.zeros((16,), jnp.float32)
        return 0
    lax.fori_loop(0, nrows, body, 0)


def _edge_phase(table_hbm, accum, src, dst, w, srcb, dstb, wb, rows, sem,
                row0, nchunk, h):
    def chunk(ci, _):
        r0 = row0 + ci * CH
        pltpu.sync_copy(src.at[pl.ds(r0, CH)], srcb)
        pltpu.sync_copy(dst.at[pl.ds(r0, CH)], dstb)
        pltpu.sync_copy(w.at[pl.ds(r0, CH)], wb)
        for j in range(CH):
            pltpu.async_copy(table_hbm.at[srcb.at[j]],
                             rows.at[pl.ds(j * LANES, LANES)], sem).wait()

        def scale(e, _):
            wv = wb[e // LANES, e % LANES]
            for o in range(0, h, 16):
                rows[e, pl.ds(o, 16)] = rows[e, pl.ds(o, 16)] * wv
            return 0
        lax.fori_loop(0, CH * LANES, scale, 0)

        for j in range(CH):
            pltpu.sync_copy(rows.at[pl.ds(j * LANES, LANES)],
                            accum.at[dstb.at[j]], add=True)
        return 0
    lax.fori_loop(0, nchunk, chunk, 0)


def _writeout(accum, out_hbm, base, nchunks):
    for j in range(nchunks):
        r = base + j * ZR
        pltpu.sync_copy(accum.at[pl.ds(r, ZR)], out_hbm.at[pl.ds(r, ZR)])


def _writeout_mean(accum, e1_hbm, e2_hbm, out_hbm, zbuf, b1, b2, base,
                   nchunks, h):
    third = jnp.float32(1.0 / 3.0)
    for j in range(nchunks):
        r = base + j * ZR
        pltpu.sync_copy(accum.at[pl.ds(r, ZR)], zbuf)
        pltpu.sync_copy(e1_hbm.at[pl.ds(r, ZR)], b1)
        pltpu.sync_copy(e2_hbm.at[pl.ds(r, ZR)], b2)

        def body(i, _):
            for o in range(0, h, 16):
                v = (zbuf[i, pl.ds(o, 16)] + b1[i, pl.ds(o, 16)]
                     + b2[i, pl.ds(o, 16)]) * third
                zbuf[i, pl.ds(o, 16)] = v
            return 0
        lax.fori_loop(0, nrows_or(ZR), body, 0)
        pltpu.sync_copy(zbuf, out_hbm.at[pl.ds(r, ZR)])


def nrows_or(x):
    return x


def _make_layer(n, h, rows_total, final):
    rpt = rows_total // NTILE
    nchunk = rpt // CH
    npt = n // NTILE
    wchunks = npt // ZR
    mesh = plsc.VectorSubcoreMesh(core_axis_name="c", subcore_axis_name="s")

    out_type = (jax.ShapeDtypeStruct((n, h), jnp.float32),
                jax.ShapeDtypeStruct((n, h), jnp.float32))

    scratch = [
        pltpu.VMEM_SHARED((n, h), jnp.float32),
        pltpu.VMEM((ZR, h), jnp.float32),
        pltpu.VMEM((ZR, h), jnp.float32),
        pltpu.VMEM((ZR, h), jnp.float32),
        pltpu.VMEM((CH, LANES), jnp.int32),
        pltpu.VMEM((CH, LANES), jnp.int32),
        pltpu.VMEM((CH, LANES), jnp.float32),
        pltpu.VMEM((CH * LANES, h), jnp.float32),
        pltpu.SemaphoreType.DMA,
    ]

    def body(*refs):
        if final:
            (tab_a, tab_b, e1a, e1b, src, dst, w, out_a, out_b,
             accum, zbuf, b1, b2, srcb, dstb, wb, rows, sem) = refs
        else:
            (tab_a, tab_b, src, dst, w, out_a, out_b,
             accum, zbuf, b1, b2, srcb, dstb, wb, rows, sem) = refs
        cid = lax.axis_index("c")
        sid = lax.axis_index("s")
        base = sid * npt
        row0 = sid * rpt

        _zero_vmem(zbuf, ZR, h)
        for j in range(wchunks):
            pltpu.sync_copy(zbuf, accum.at[pl.ds(base + j * ZR, ZR)])
        plsc.subcore_barrier()

        @pl.when(cid == 0)
        def _():
            _edge_phase(tab_a, accum, src, dst, w, srcb, dstb, wb, rows,
                        sem, row0, nchunk, h)

        @pl.when(cid == 1)
        def _():
            _edge_phase(tab_b, accum, src, dst, w, srcb, dstb, wb, rows,
                        sem, row0, nchunk, h)
        plsc.subcore_barrier()

        if final:
            @pl.when(cid == 0)
            def _():
                _writeout_mean(accum, e1a, tab_a, out_a, zbuf, b1, b2,
                               base, wchunks, h)

            @pl.when(cid == 1)
            def _():
                _writeout_mean(accum, e1b, tab_b, out_b, zbuf, b1, b2,
                               base, wchunks, h)
        else:
            @pl.when(cid == 0)
            def _():
                _writeout(accum, out_a, base, wchunks)

            @pl.when(cid == 1)
            def _():
                _writeout(accum, out_b, base, wchunks)

    return pl.kernel(body, mesh=mesh, out_type=out_type,
                     scratch_types=scratch)


@jax.jit
def kernel(user_emb, item_emb, edge_index, edge_weight):
    n_user = user_emb.shape[0]
    n = n_user + item_emb.shape[0]
    emb = user_emb.shape[1]
    h = emb // 2
    e = edge_weight.shape[0]

    ego = jnp.concatenate([user_emb, item_emb], axis=0)
    ego_a = ego[:, :h]
    ego_b = ego[:, h:]

    quant = NTILE * CH * LANES
    e_pad = -(-e // quant) * quant
    src = jnp.pad(edge_index[1].astype(jnp.int32), (0, e_pad - e))
    dst = jnp.pad(edge_index[0].astype(jnp.int32), (0, e_pad - e))
    w = jnp.pad(edge_weight, (0, e_pad - e))
    rows_total = e_pad // LANES
    src = src.reshape(rows_total, LANES)
    dst = dst.reshape(rows_total, LANES)
    w = w.reshape(rows_total, LANES)

    layer = _make_layer(n, h, rows_total, final=False)
    layer_fin = _make_layer(n, h, rows_total, final=True)

    e1a, e1b = layer(ego_a, ego_b, src, dst, w)
    e2a, e2b = layer(e1a, e1b, src, dst, w)
    fa, fb = layer_fin(e2a, e2b, e1a, e1b, src, dst, w)

    fin = jnp.concatenate([fa, fb], axis=1)
    return fin[:n_user], fin[n_user:]


# speedup vs baseline: 4.6810x; 4.6810x over previous
import jax
import jax.numpy as jnp
from jax import lax
from jax.experimental import pallas as pl
from jax.experimental.pallas import tpu as pltpu
from jax.experimental.pallas import tpu_sc as plsc

NTILE = 16
NCORE = 2
LANES = 128
CH = 4
ZR = 80


def _zero_vmem(buf, nrows, h):
    def body(i, _):
        for o in range(0, h, 16):
            buf[i, pl.ds(o, 16)] = jnp.zeros((16,), jnp.float32)
        return 0
    lax.fori_loop(0, nrows, body, 0)


def _edge_phase(table_hbm, accum, src, dst, w, srcb, dstb, wb, rows, sem,
                row0, nchunk, h):
    def chunk(ci, _):
        r0 = row0 + ci * CH
        pltpu.sync_copy(src.at[pl.ds(r0, CH)], srcb)
        pltpu.sync_copy(dst.at[pl.ds(r0, CH)], dstb)
        pltpu.sync_copy(w.at[pl.ds(r0, CH)], wb)
        for j in range(CH):
            pltpu.async_copy(table_hbm.at[srcb.at[j]],
                             rows.at[pl.ds(j * LANES, LANES)], sem).wait()

        def scale(g, _):
            wv = wb[g // (LANES // 16), pl.ds((g % (LANES // 16)) * 16, 16)]
            for i in range(16):
                wi = wv[i]
                e = g * 16 + i
                for o in range(0, h, 16):
                    rows[e, pl.ds(o, 16)] = rows[e, pl.ds(o, 16)] * wi
            return 0
        lax.fori_loop(0, CH * LANES // 16, scale, 0)

        for j in range(CH):
            pltpu.sync_copy(rows.at[pl.ds(j * LANES, LANES)],
                            accum.at[dstb.at[j]], add=True)
        return 0
    lax.fori_loop(0, nchunk, chunk, 0)


def _writeout(accum, out_hbm, sid, nchunks):
    for k in range((nchunks + NTILE - 1) // NTILE):
        c = sid + k * NTILE

        @pl.when(c < nchunks)
        def _():
            r = c * ZR
            pltpu.sync_copy(accum.at[pl.ds(r, ZR)], out_hbm.at[pl.ds(r, ZR)])


def _writeout_mean(accum, e1_hbm, e2_hbm, out_hbm, zbuf, b1, b2, sid,
                   nchunks, h):
    third = jnp.float32(1.0 / 3.0)
    for k in range((nchunks + NTILE - 1) // NTILE):
        c = sid + k * NTILE

        @pl.when(c < nchunks)
        def _():
            r = c * ZR
            pltpu.sync_copy(accum.at[pl.ds(r, ZR)], zbuf)
            pltpu.sync_copy(e1_hbm.at[pl.ds(r, ZR)], b1)
            pltpu.sync_copy(e2_hbm.at[pl.ds(r, ZR)], b2)

            def body(i, _):
                for o in range(0, h, 16):
                    v = (zbuf[i, pl.ds(o, 16)] + b1[i, pl.ds(o, 16)]
                         + b2[i, pl.ds(o, 16)]) * third
                    zbuf[i, pl.ds(o, 16)] = v
                return 0
            lax.fori_loop(0, ZR, body, 0)
            pltpu.sync_copy(zbuf, out_hbm.at[pl.ds(r, ZR)])


def _make_layer(n, h, rows_total, final):
    rpt = rows_total // NTILE
    nchunk = rpt // CH
    wchunks = n // ZR
    mesh = plsc.VectorSubcoreMesh(core_axis_name="c", subcore_axis_name="s")

    out_type = (jax.ShapeDtypeStruct((n, h), jnp.float32),
                jax.ShapeDtypeStruct((n, h), jnp.float32))

    scratch = [
        pltpu.VMEM_SHARED((n, h), jnp.float32),
        pltpu.VMEM((ZR, h), jnp.float32),
        pltpu.VMEM((ZR, h), jnp.float32),
        pltpu.VMEM((ZR, h), jnp.float32),
        pltpu.VMEM((CH, LANES), jnp.int32),
        pltpu.VMEM((CH, LANES), jnp.int32),
        pltpu.VMEM((CH, LANES), jnp.float32),
        pltpu.VMEM((CH * LANES, h), jnp.float32),
        pltpu.SemaphoreType.DMA,
    ]

    def body(*refs):
        if final:
            (tab_a, tab_b, e1a, e1b, src, dst, w, out_a, out_b,
             accum, zbuf, b1, b2, srcb, dstb, wb, rows, sem) = refs
        else:
            (tab_a, tab_b, src, dst, w, out_a, out_b,
             accum, zbuf, b1, b2, srcb, dstb, wb, rows, sem) = refs
        cid = lax.axis_index("c")
        sid = lax.axis_index("s")
        row0 = sid * rpt

        _zero_vmem(zbuf, ZR, h)
        for k in range((wchunks + NTILE - 1) // NTILE):
            c = sid + k * NTILE

            @pl.when(c < wchunks)
            def _():
                pltpu.sync_copy(zbuf, accum.at[pl.ds(c * ZR, ZR)])
        plsc.subcore_barrier()

        @pl.when(cid == 0)
        def _():
            _edge_phase(tab_a, accum, src, dst, w, srcb, dstb, wb, rows,
                        sem, row0, nchunk, h)

        @pl.when(cid == 1)
        def _():
            _edge_phase(tab_b, accum, src, dst, w, srcb, dstb, wb, rows,
                        sem, row0, nchunk, h)
        plsc.subcore_barrier()

        if final:
            @pl.when(cid == 0)
            def _():
                _writeout_mean(accum, e1a, tab_a, out_a, zbuf, b1, b2,
                               sid, wchunks, h)

            @pl.when(cid == 1)
            def _():
                _writeout_mean(accum, e1b, tab_b, out_b, zbuf, b1, b2,
                               sid, wchunks, h)
        else:
            @pl.when(cid == 0)
            def _():
                _writeout(accum, out_a, sid, wchunks)

            @pl.when(cid == 1)
            def _():
                _writeout(accum, out_b, sid, wchunks)

    return pl.kernel(body, mesh=mesh, out_type=out_type,
                     scratch_types=scratch,
                     compiler_params=pltpu.CompilerParams(
                         use_tc_tiling_on_sc=False))


@jax.jit
def kernel(user_emb, item_emb, edge_index, edge_weight):
    n_user = user_emb.shape[0]
    n = n_user + item_emb.shape[0]
    emb = user_emb.shape[1]
    h = emb // 2
    e = edge_weight.shape[0]

    ego = jnp.concatenate([user_emb, item_emb], axis=0)
    ego_a = ego[:, :h]
    ego_b = ego[:, h:]

    quant = NTILE * CH * LANES
    e_pad = -(-e // quant) * quant
    src = jnp.pad(edge_index[1].astype(jnp.int32), (0, e_pad - e))
    dst = jnp.pad(edge_index[0].astype(jnp.int32), (0, e_pad - e))
    w = jnp.pad(edge_weight, (0, e_pad - e))
    rows_total = e_pad // LANES
    src = src.reshape(rows_total, LANES)
    dst = dst.reshape(rows_total, LANES)
    w = w.reshape(rows_total, LANES)

    layer = _make_layer(n, h, rows_total, final=False)
    layer_fin = _make_layer(n, h, rows_total, final=True)

    e1a, e1b = layer(ego_a, ego_b, src, dst, w)
    e2a, e2b = layer(e1a, e1b, src, dst, w)
    fa, fb = layer_fin(e2a, e2b, e1a, e1b, src, dst, w)

    fin = jnp.concatenate([fa, fb], axis=1)
    return fin[:n_user], fin[n_user:]


# speedup vs baseline: 5.3348x; 1.1397x over previous
import jax
import jax.numpy as jnp
from jax import lax
from jax.experimental import pallas as pl
from jax.experimental.pallas import tpu as pltpu
from jax.experimental.pallas import tpu_sc as plsc

NTILE = 16
LANES = 128
HB = 2
HE = HB * LANES
ZR = 80


def _scale(rows, wb, base):
    def body(g, _):
        wv = wb[g // (LANES // 16), pl.ds((g % (LANES // 16)) * 16, 16)]
        for i in range(16):
            wi = wv[i]
            e = base + g * 16 + i
            rows[e, pl.ds(0, 16)] = rows[e, pl.ds(0, 16)] * wi
            rows[e, pl.ds(16, 16)] = rows[e, pl.ds(16, 16)] * wi
        return 0
    lax.fori_loop(0, HE // 16, body, 0)


def _edge_phase(tab, accum, src, dst, w, rows,
                sb0, db0, wb0, sb1, db1, wb1, gs0, gs1, ss0, ss1,
                row0, nhalf):
    def load_idx(sb, db, wb, r):
        pltpu.sync_copy(src.at[pl.ds(r, HB)], sb)
        pltpu.sync_copy(dst.at[pl.ds(r, HB)], db)
        pltpu.sync_copy(w.at[pl.ds(r, HB)], wb)

    def fire_gather(sb, gsem, base):
        for j in range(HB):
            pltpu.async_copy(tab.at[sb.at[j]],
                             rows.at[pl.ds(base + j * LANES, LANES)], gsem)

    def wait_gather(sb, gsem, base):
        for j in range(HB):
            pltpu.make_async_copy(
                tab.at[sb.at[j]],
                rows.at[pl.ds(base + j * LANES, LANES)], gsem).wait()

    def fire_scatter(db, ssem, base):
        for j in range(HB):
            pltpu.async_copy(rows.at[pl.ds(base + j * LANES, LANES)],
                             accum.at[db.at[j]], ssem, add=True)

    def wait_scatter(db, ssem, base):
        for j in range(HB):
            pltpu.make_async_copy(rows.at[pl.ds(base + j * LANES, LANES)],
                                  accum.at[db.at[j]], ssem).wait()

    load_idx(sb0, db0, wb0, row0)
    fire_gather(sb0, gs0, 0)

    def pair(p, _):
        r1 = row0 + (2 * p + 1) * HB
        r2 = row0 + (2 * p + 2) * HB

        wait_gather(sb0, gs0, 0)

        @pl.when(p > 0)
        def _():
            wait_scatter(db1, ss1, HE)
        load_idx(sb1, db1, wb1, r1)
        fire_gather(sb1, gs1, HE)

        _scale(rows, wb0, 0)
        fire_scatter(db0, ss0, 0)

        wait_gather(sb1, gs1, HE)
        _scale(rows, wb1, HE)
        fire_scatter(db1, ss1, HE)

        wait_scatter(db0, ss0, 0)
        load_idx(sb0, db0, wb0, r2)
        fire_gather(sb0, gs0, 0)
        return 0

    lax.fori_loop(0, nhalf // 2, pair, 0)
    wait_gather(sb0, gs0, 0)
    wait_scatter(db1, ss1, HE)


def _writeout(accum, out_hbm, sid, nchunks):
    for k in range((nchunks + NTILE - 1) // NTILE):
        c = sid + k * NTILE

        @pl.when(c < nchunks)
        def _():
            r = c * ZR
            pltpu.sync_copy(accum.at[pl.ds(r, ZR)], out_hbm.at[pl.ds(r, ZR)])


def _writeout_mean(accum, e1_hbm, e2_hbm, out_hbm, rows, sid, nchunks):
    ba, b1, b2 = 0, 128, 256
    third = jnp.float32(1.0 / 3.0)
    for k in range((nchunks + NTILE - 1) // NTILE):
        c = sid + k * NTILE

        @pl.when(c < nchunks)
        def _():
            r = c * ZR
            pltpu.sync_copy(accum.at[pl.ds(r, ZR)], rows.at[pl.ds(ba, ZR)])
            pltpu.sync_copy(e1_hbm.at[pl.ds(r, ZR)], rows.at[pl.ds(b1, ZR)])
            pltpu.sync_copy(e2_hbm.at[pl.ds(r, ZR)], rows.at[pl.ds(b2, ZR)])

            def body(i, _):
                for o in range(0, 32, 16):
                    v = (rows[ba + i, pl.ds(o, 16)]
                         + rows[b1 + i, pl.ds(o, 16)]
                         + rows[b2 + i, pl.ds(o, 16)]) * third
                    rows[ba + i, pl.ds(o, 16)] = v
                return 0
            lax.fori_loop(0, ZR, body, 0)
            pltpu.sync_copy(rows.at[pl.ds(ba, ZR)], out_hbm.at[pl.ds(r, ZR)])


def _make_layer(n, h, rows_total, final):
    rpt = rows_total // NTILE
    nhalf = rpt // HB
    wchunks = n // ZR
    mesh = plsc.VectorSubcoreMesh(core_axis_name="c", subcore_axis_name="s")

    out_type = (jax.ShapeDtypeStruct((n, h), jnp.float32),
                jax.ShapeDtypeStruct((n, h), jnp.float32))

    scratch = [
        pltpu.VMEM_SHARED((n, h), jnp.float32),
        pltpu.VMEM((2 * HE, h), jnp.float32),
        pltpu.VMEM((HB, LANES), jnp.int32),
        pltpu.VMEM((HB, LANES), jnp.int32),
        pltpu.VMEM((HB, LANES), jnp.float32),
        pltpu.VMEM((HB, LANES), jnp.int32),
        pltpu.VMEM((HB, LANES), jnp.int32),
        pltpu.VMEM((HB, LANES), jnp.float32),
        pltpu.SemaphoreType.DMA,
        pltpu.SemaphoreType.DMA,
        pltpu.SemaphoreType.DMA,
        pltpu.SemaphoreType.DMA,
    ]

    def body(*refs):
        if final:
            (tab_a, tab_b, e1a, e1b, src, dst, w, out_a, out_b, accum, rows,
             sb0, db0, wb0, sb1, db1, wb1, gs0, gs1, ss0, ss1) = refs
        else:
            (tab_a, tab_b, src, dst, w, out_a, out_b, accum, rows,
             sb0, db0, wb0, sb1, db1, wb1, gs0, gs1, ss0, ss1) = refs
        cid = lax.axis_index("c")
        sid = lax.axis_index("s")
        row0 = sid * rpt

        def zb(i, _):
            rows[i, pl.ds(0, 16)] = jnp.zeros((16,), jnp.float32)
            rows[i, pl.ds(16, 16)] = jnp.zeros((16,), jnp.float32)
            return 0
        lax.fori_loop(0, ZR, zb, 0)
        for k in range((wchunks + NTILE - 1) // NTILE):
            c = sid + k * NTILE

            @pl.when(c < wchunks)
            def _():
                pltpu.sync_copy(rows.at[pl.ds(0, ZR)],
                                accum.at[pl.ds(c * ZR, ZR)])
        plsc.subcore_barrier()

        @pl.when(cid == 0)
        def _():
            _edge_phase(tab_a, accum, src, dst, w, rows, sb0, db0, wb0,
                        sb1, db1, wb1, gs0, gs1, ss0, ss1, row0, nhalf)

        @pl.when(cid == 1)
        def _():
            _edge_phase(tab_b, accum, src, dst, w, rows, sb0, db0, wb0,
                        sb1, db1, wb1, gs0, gs1, ss0, ss1, row0, nhalf)
        plsc.subcore_barrier()

        if final:
            @pl.when(cid == 0)
            def _():
                _writeout_mean(accum, e1a, tab_a, out_a, rows, sid, wchunks)

            @pl.when(cid == 1)
            def _():
                _writeout_mean(accum, e1b, tab_b, out_b, rows, sid, wchunks)
        else:
            @pl.when(cid == 0)
            def _():
                _writeout(accum, out_a, sid, wchunks)

            @pl.when(cid == 1)
            def _():
                _writeout(accum, out_b, sid, wchunks)

    return pl.kernel(body, mesh=mesh, out_type=out_type,
                     scratch_types=scratch,
                     compiler_params=pltpu.CompilerParams(
                         use_tc_tiling_on_sc=False))


@jax.jit
def kernel(user_emb, item_emb, edge_index, edge_weight):
    n_user = user_emb.shape[0]
    n = n_user + item_emb.shape[0]
    emb = user_emb.shape[1]
    h = emb // 2
    e = edge_weight.shape[0]

    ego = jnp.concatenate([user_emb, item_emb], axis=0)
    ego_a = ego[:, :h]
    ego_b = ego[:, h:]

    quant = NTILE * 2 * HB * LANES
    e_pad = -(-e // quant) * quant
    slack = HB * LANES
    src = jnp.pad(edge_index[1].astype(jnp.int32), (0, e_pad + slack - e))
    dst = jnp.pad(edge_index[0].astype(jnp.int32), (0, e_pad + slack - e))
    w = jnp.pad(edge_weight, (0, e_pad + slack - e))
    rows_total = e_pad // LANES
    src = src.reshape(rows_total + HB, LANES)
    dst = dst.reshape(rows_total + HB, LANES)
    w = w.reshape(rows_total + HB, LANES)

    layer = _make_layer(n, h, rows_total, final=False)
    layer_fin = _make_layer(n, h, rows_total, final=True)

    e1a, e1b = layer(ego_a, ego_b, src, dst, w)
    e2a, e2b = layer(e1a, e1b, src, dst, w)
    fa, fb = layer_fin(e2a, e2b, e1a, e1b, src, dst, w)

    fin = jnp.concatenate([fa, fb], axis=1)
    return fin[:n_user], fin[n_user:]


# speedup vs baseline: 6.3035x; 1.1816x over previous
import jax
import jax.numpy as jnp
from jax import lax
from jax.experimental import pallas as pl
from jax.experimental.pallas import tpu as pltpu
from jax.experimental.pallas import tpu_sc as plsc

NTILE = 16
LANES = 128
HB = 2
HE = HB * LANES
GR = 8
ZR = 80


def _scale(rows, wb, k, base):
    def body(g, _):
        wv = wb[k * HB + g // (LANES // 16),
                pl.ds((g % (LANES // 16)) * 16, 16)]
        for i in range(16):
            wi = wv[i]
            e = base + g * 16 + i
            rows[e, pl.ds(0, 16)] = rows[e, pl.ds(0, 16)] * wi
            rows[e, pl.ds(16, 16)] = rows[e, pl.ds(16, 16)] * wi
        return 0
    lax.fori_loop(0, HE // 16, body, 0)


def _edge_phase(tab, accum, src, dst, w, rows,
                sbA, dbA, wbA, sbB, dbB, wbB, gs0, gs1, ss0, ss1,
                isA, isB, row0, niter):
    def fire_idx(sb, db, wb, isem, r):
        pltpu.async_copy(src.at[pl.ds(r, GR)], sb, isem)
        pltpu.async_copy(dst.at[pl.ds(r, GR)], db, isem)
        pltpu.async_copy(w.at[pl.ds(r, GR)], wb, isem)

    def wait_idx(sb, db, wb, isem, r):
        pltpu.make_async_copy(src.at[pl.ds(r, GR)], sb, isem).wait()
        pltpu.make_async_copy(dst.at[pl.ds(r, GR)], db, isem).wait()
        pltpu.make_async_copy(w.at[pl.ds(r, GR)], wb, isem).wait()

    def fire_gather(sb, k, gsem, base):
        for j in range(HB):
            pltpu.async_copy(tab.at[sb.at[k * HB + j]],
                             rows.at[pl.ds(base + j * LANES, LANES)], gsem)

    def wait_gather(sb, k, gsem, base):
        for j in range(HB):
            pltpu.make_async_copy(
                tab.at[sb.at[k * HB + j]],
                rows.at[pl.ds(base + j * LANES, LANES)], gsem).wait()

    def fire_scatter(db, k, ssem, base):
        for j in range(HB):
            pltpu.async_copy(rows.at[pl.ds(base + j * LANES, LANES)],
                             accum.at[db.at[k * HB + j]], ssem, add=True)

    def wait_scatter(db, k, ssem, base):
        for j in range(HB):
            pltpu.make_async_copy(
                rows.at[pl.ds(base + j * LANES, LANES)],
                accum.at[db.at[k * HB + j]], ssem).wait()

    fire_idx(sbA, dbA, wbA, isA, row0)
    wait_idx(sbA, dbA, wbA, isA, row0)
    fire_gather(sbA, 0, gs0, 0)

    halves_per_group = GR // HB

    def iteration(it, _):
        rA = row0 + it * 2 * GR
        rB = rA + GR
        rA2 = rA + 2 * GR

        wait_gather(sbA, 0, gs0, 0)

        @pl.when(it > 0)
        def _():
            wait_scatter(dbB, 3, ss1, HE)
        fire_gather(sbA, 1, gs1, HE)
        _scale(rows, wbA, 0, 0)
        fire_scatter(dbA, 0, ss0, 0)
        fire_idx(sbB, dbB, wbB, isB, rB)

        wait_gather(sbA, 1, gs1, HE)
        wait_scatter(dbA, 0, ss0, 0)
        fire_gather(sbA, 2, gs0, 0)
        _scale(rows, wbA, 1, HE)
        fire_scatter(dbA, 1, ss1, HE)

        wait_gather(sbA, 2, gs0, 0)
        wait_scatter(dbA, 1, ss1, HE)
        fire_gather(sbA, 3, gs1, HE)
        _scale(rows, wbA, 2, 0)
        fire_scatter(dbA, 2, ss0, 0)

        wait_gather(sbA, 3, gs1, HE)
        wait_scatter(dbA, 2, ss0, 0)
        wait_idx(sbB, dbB, wbB, isB, rB)
        fire_gather(sbB, 0, gs0, 0)
        _scale(rows, wbA, 3, HE)
        fire_scatter(dbA, 3, ss1, HE)

        wait_gather(sbB, 0, gs0, 0)
        wait_scatter(dbA, 3, ss1, HE)
        fire_gather(sbB, 1, gs1, HE)
        _scale(rows, wbB, 0, 0)
        fire_scatter(dbB, 0, ss0, 0)
        fire_idx(sbA, dbA, wbA, isA, rA2)

        wait_gather(sbB, 1, gs1, HE)
        wait_scatter(dbB, 0, ss0, 0)
        fire_gather(sbB, 2, gs0, 0)
        _scale(rows, wbB, 1, HE)
        fire_scatter(dbB, 1, ss1, HE)

        wait_gather(sbB, 2, gs0, 0)
        wait_scatter(dbB, 1, ss1, HE)
        fire_gather(sbB, 3, gs1, HE)
        _scale(rows, wbB, 2, 0)
        fire_scatter(dbB, 2, ss0, 0)

        wait_gather(sbB, 3, gs1, HE)
        wait_scatter(dbB, 2, ss0, 0)
        wait_idx(sbA, dbA, wbA, isA, rA2)
        fire_gather(sbA, 0, gs0, 0)
        _scale(rows, wbB, 3, HE)
        fire_scatter(dbB, 3, ss1, HE)
        return 0

    lax.fori_loop(0, niter, iteration, 0)
    wait_gather(sbA, 0, gs0, 0)
    wait_scatter(dbB, 3, ss1, HE)


def _writeout(accum, out_hbm, sid, nchunks):
    for k in range((nchunks + NTILE - 1) // NTILE):
        c = sid + k * NTILE

        @pl.when(c < nchunks)
        def _():
            r = c * ZR
            pltpu.sync_copy(accum.at[pl.ds(r, ZR)], out_hbm.at[pl.ds(r, ZR)])


def _writeout_mean(accum, e1_hbm, e2_hbm, out_hbm, rows, sid, nchunks):
    ba, b1, b2 = 0, 128, 256
    third = jnp.float32(1.0 / 3.0)
    for k in range((nchunks + NTILE - 1) // NTILE):
        c = sid + k * NTILE

        @pl.when(c < nchunks)
        def _():
            r = c * ZR
            pltpu.sync_copy(accum.at[pl.ds(r, ZR)], rows.at[pl.ds(ba, ZR)])
            pltpu.sync_copy(e1_hbm.at[pl.ds(r, ZR)], rows.at[pl.ds(b1, ZR)])
            pltpu.sync_copy(e2_hbm.at[pl.ds(r, ZR)], rows.at[pl.ds(b2, ZR)])

            def body(i, _):
                for o in range(0, 32, 16):
                    v = (rows[ba + i, pl.ds(o, 16)]
                         + rows[b1 + i, pl.ds(o, 16)]
                         + rows[b2 + i, pl.ds(o, 16)]) * third
                    rows[ba + i, pl.ds(o, 16)] = v
                return 0
            lax.fori_loop(0, ZR, body, 0)
            pltpu.sync_copy(rows.at[pl.ds(ba, ZR)], out_hbm.at[pl.ds(r, ZR)])


def _make_layer(n, h, rows_total, final):
    rpt = rows_total // NTILE
    niter = rpt // (2 * GR)
    wchunks = n // ZR
    mesh = plsc.VectorSubcoreMesh(core_axis_name="c", subcore_axis_name="s")

    out_type = (jax.ShapeDtypeStruct((n, h), jnp.float32),
                jax.ShapeDtypeStruct((n, h), jnp.float32))

    scratch = [
        pltpu.VMEM_SHARED((n, h), jnp.float32),
        pltpu.VMEM((2 * HE, h), jnp.float32),
        pltpu.VMEM((GR, LANES), jnp.int32),
        pltpu.VMEM((GR, LANES), jnp.int32),
        pltpu.VMEM((GR, LANES), jnp.float32),
        pltpu.VMEM((GR, LANES), jnp.int32),
        pltpu.VMEM((GR, LANES), jnp.int32),
        pltpu.VMEM((GR, LANES), jnp.float32),
        pltpu.SemaphoreType.DMA,
        pltpu.SemaphoreType.DMA,
        pltpu.SemaphoreType.DMA,
        pltpu.SemaphoreType.DMA,
        pltpu.SemaphoreType.DMA,
        pltpu.SemaphoreType.DMA,
    ]

    def body(*refs):
        if final:
            (tab_a, tab_b, e1a, e1b, src, dst, w, out_a, out_b, accum, rows,
             sbA, dbA, wbA, sbB, dbB, wbB,
             gs0, gs1, ss0, ss1, isA, isB) = refs
        else:
            (tab_a, tab_b, src, dst, w, out_a, out_b, accum, rows,
             sbA, dbA, wbA, sbB, dbB, wbB,
             gs0, gs1, ss0, ss1, isA, isB) = refs
        cid = lax.axis_index("c")
        sid = lax.axis_index("s")
        row0 = sid * rpt

        def zb(i, _):
            rows[i, pl.ds(0, 16)] = jnp.zeros((16,), jnp.float32)
            rows[i, pl.ds(16, 16)] = jnp.zeros((16,), jnp.float32)
            return 0
        lax.fori_loop(0, ZR, zb, 0)
        for k in range((wchunks + NTILE - 1) // NTILE):
            c = sid + k * NTILE

            @pl.when(c < wchunks)
            def _():
                pltpu.sync_copy(rows.at[pl.ds(0, ZR)],
                                accum.at[pl.ds(c * ZR, ZR)])
        plsc.subcore_barrier()

        @pl.when(cid == 0)
        def _():
            _edge_phase(tab_a, accum, src, dst, w, rows, sbA, dbA, wbA,
                        sbB, dbB, wbB, gs0, gs1, ss0, ss1, isA, isB,
                        row0, niter)

        @pl.when(cid == 1)
        def _():
            _edge_phase(tab_b, accum, src, dst, w, rows, sbA, dbA, wbA,
                        sbB, dbB, wbB, gs0, gs1, ss0, ss1, isA, isB,
                        row0, niter)
        plsc.subcore_barrier()

        if final:
            @pl.when(cid == 0)
            def _():
                _writeout_mean(accum, e1a, tab_a, out_a, rows, sid, wchunks)

            @pl.when(cid == 1)
            def _():
                _writeout_mean(accum, e1b, tab_b, out_b, rows, sid, wchunks)
        else:
            @pl.when(cid == 0)
            def _():
                _writeout(accum, out_a, sid, wchunks)

            @pl.when(cid == 1)
            def _():
                _writeout(accum, out_b, sid, wchunks)

    return pl.kernel(body, mesh=mesh, out_type=out_type,
                     scratch_types=scratch,
                     compiler_params=pltpu.CompilerParams(
                         use_tc_tiling_on_sc=False))


@jax.jit
def kernel(user_emb, item_emb, edge_index, edge_weight):
    n_user = user_emb.shape[0]
    n = n_user + item_emb.shape[0]
    emb = user_emb.shape[1]
    h = emb // 2
    e = edge_weight.shape[0]

    ego = jnp.concatenate([user_emb, item_emb], axis=0)
    ego_a = ego[:, :h]
    ego_b = ego[:, h:]

    quant = NTILE * 2 * GR * LANES
    e_pad = -(-e // quant) * quant
    slack = GR * LANES
    src = jnp.pad(edge_index[1].astype(jnp.int32), (0, e_pad + slack - e))
    dst = jnp.pad(edge_index[0].astype(jnp.int32), (0, e_pad + slack - e))
    w = jnp.pad(edge_weight, (0, e_pad + slack - e))
    rows_total = e_pad // LANES
    src = src.reshape(rows_total + GR, LANES)
    dst = dst.reshape(rows_total + GR, LANES)
    w = w.reshape(rows_total + GR, LANES)

    layer = _make_layer(n, h, rows_total, final=False)
    layer_fin = _make_layer(n, h, rows_total, final=True)

    e1a, e1b = layer(ego_a, ego_b, src, dst, w)
    e2a, e2b = layer(e1a, e1b, src, dst, w)
    fa, fb = layer_fin(e2a, e2b, e1a, e1b, src, dst, w)

    fin = jnp.concatenate([fa, fb], axis=1)
    return fin[:n_user], fin[n_user:]


# speedup vs baseline: 6.9897x; 1.1089x over previous
import jax
import jax.numpy as jnp
from jax import lax
from jax.experimental import pallas as pl
from jax.experimental.pallas import tpu as pltpu
from jax.experimental.pallas import tpu_sc as plsc

NTILE = 16
LANES = 128
RING = 5
GLEAD = 3
SLAG = 2
GR = 10
UNROLL = 2 * GR
ZR = 80


def _scale(rows, wb, slot, ring):
    base = ring * LANES
    lanes = lax.iota(jnp.int32, 16)

    def body(g, _):
        wv = wb[slot, pl.ds((g // 2) * 16, 16)]
        sh = (g & 1) * 8
        ws = wv[(lanes + sh) & 15]
        for i in range(8):
            wi = ws[i]
            e = base + g * 8 + i
            rows[e, pl.ds(0, 16)] = rows[e, pl.ds(0, 16)] * wi
            rows[e, pl.ds(16, 16)] = rows[e, pl.ds(16, 16)] * wi
        return 0
    lax.fori_loop(0, 2 * (LANES // 16), body, 0)


def _edge_phase(tab, accum, src, dst, w, rows,
                sbA, dbA, wbA, sbB, dbB, wbB, gsems, ssems, isA, isB,
                row0, niter):
    def idx_slot(kk):
        if kk < GR:
            return sbA, dbA, wbA, kk
        elif kk < 2 * GR:
            return sbB, dbB, wbB, kk - GR
        else:
            return sbA, dbA, wbA, kk - 2 * GR

    def fire_gather(kk):
        sb, _, _, slot = idx_slot(kk)
        r = kk % RING
        pltpu.async_copy(tab.at[sb.at[slot]],
                         rows.at[pl.ds(r * LANES, LANES)], gsems[r])

    def wait_gather(kk):
        sb, _, _, slot = idx_slot(kk)
        r = kk % RING
        pltpu.make_async_copy(tab.at[sb.at[slot]],
                              rows.at[pl.ds(r * LANES, LANES)],
                              gsems[r]).wait()

    def fire_scatter(kk):
        _, db, _, slot = idx_slot(kk)
        r = kk % RING
        pltpu.async_copy(rows.at[pl.ds(r * LANES, LANES)],
                         accum.at[db.at[slot]], ssems[r], add=True)

    def wait_scatter(kk):
        _, db, _, slot = idx_slot(kk)
        r = kk % RING
        pltpu.make_async_copy(rows.at[pl.ds(r * LANES, LANES)],
                              accum.at[db.at[slot]], ssems[r]).wait()

    def fire_idx(sb, db, wb, isem, r):
        pltpu.async_copy(src.at[pl.ds(r, GR)], sb, isem)
        pltpu.async_copy(dst.at[pl.ds(r, GR)], db, isem)
        pltpu.async_copy(w.at[pl.ds(r, GR)], wb, isem)

    def wait_idx(sb, db, wb, isem, r):
        pltpu.make_async_copy(src.at[pl.ds(r, GR)], sb, isem).wait()
        pltpu.make_async_copy(dst.at[pl.ds(r, GR)], db, isem).wait()
        pltpu.make_async_copy(w.at[pl.ds(r, GR)], wb, isem).wait()

    fire_idx(sbA, dbA, wbA, isA, row0)
    wait_idx(sbA, dbA, wbA, isA, row0)
    for kk in range(GLEAD):
        fire_gather(kk)

    def iteration(it, _):
        rbase = row0 + it * UNROLL
        for k in range(UNROLL):
            wait_gather(k)
            wb = wbA if k < GR else wbB
            _scale(rows, wb, k % GR, k % RING)
            fire_scatter(k)
            if k < SLAG:
                @pl.when(it > 0)
                def _():
                    wait_scatter(k + UNROLL - SLAG)
            else:
                wait_scatter(k - SLAG)
            fire_gather(k + GLEAD)
            if k == 2:
                fire_idx(sbB, dbB, wbB, isB, rbase + GR)
            elif k == GR - 3:
                wait_idx(sbB, dbB, wbB, isB, rbase + GR)
            elif k == GR + 2:
                fire_idx(sbA, dbA, wbA, isA, rbase + UNROLL)
            elif k == UNROLL - 3:
                wait_idx(sbA, dbA, wbA, isA, rbase + UNROLL)
        return 0

    lax.fori_loop(0, niter, iteration, 0)
    for kk in range(GLEAD):
        wait_gather(2 * GR + kk)
    for kk in range(SLAG):
        wait_scatter(UNROLL - SLAG + kk)


def _writeout(accum, out_hbm, sid, nchunks):
    def body(k, _):
        c = sid + k * NTILE

        @pl.when(c < nchunks)
        def _():
            r = c * ZR
            pltpu.sync_copy(accum.at[pl.ds(r, ZR)], out_hbm.at[pl.ds(r, ZR)])
        return 0
    lax.fori_loop(0, (nchunks + NTILE - 1) // NTILE, body, 0)


def _writeout_mean(accum, e1_hbm, e2_hbm, out_hbm, rows, sid, nchunks):
    ba, b1, b2 = 0, 128, 256
    third = jnp.float32(1.0 / 3.0)

    def chunk(k, _):
        c = sid + k * NTILE

        @pl.when(c < nchunks)
        def _():
            r = c * ZR
            pltpu.sync_copy(accum.at[pl.ds(r, ZR)], rows.at[pl.ds(ba, ZR)])
            pltpu.sync_copy(e1_hbm.at[pl.ds(r, ZR)], rows.at[pl.ds(b1, ZR)])
            pltpu.sync_copy(e2_hbm.at[pl.ds(r, ZR)], rows.at[pl.ds(b2, ZR)])

            def body(i, _):
                for o in range(0, 32, 16):
                    v = (rows[ba + i, pl.ds(o, 16)]
                         + rows[b1 + i, pl.ds(o, 16)]
                         + rows[b2 + i, pl.ds(o, 16)]) * third
                    rows[ba + i, pl.ds(o, 16)] = v
                return 0
            lax.fori_loop(0, ZR, body, 0)
            pltpu.sync_copy(rows.at[pl.ds(ba, ZR)], out_hbm.at[pl.ds(r, ZR)])
        return 0
    lax.fori_loop(0, (nchunks + NTILE - 1) // NTILE, chunk, 0)


def _make_layer(n, h, rows_total, final):
    rpt = rows_total // NTILE
    niter = rpt // UNROLL
    wchunks = n // ZR
    mesh = plsc.VectorSubcoreMesh(core_axis_name="c", subcore_axis_name="s")

    out_type = (jax.ShapeDtypeStruct((n, h), jnp.float32),
                jax.ShapeDtypeStruct((n, h), jnp.float32))

    scratch = [
        pltpu.VMEM_SHARED((n, h), jnp.float32),
        pltpu.VMEM((RING * LANES, h), jnp.float32),
        pltpu.VMEM((GR, LANES), jnp.int32),
        pltpu.VMEM((GR, LANES), jnp.int32),
        pltpu.VMEM((GR, LANES), jnp.float32),
        pltpu.VMEM((GR, LANES), jnp.int32),
        pltpu.VMEM((GR, LANES), jnp.int32),
        pltpu.VMEM((GR, LANES), jnp.float32),
    ] + [pltpu.SemaphoreType.DMA] * (2 * RING + 2)

    def body(*refs):
        if final:
            (tab_a, tab_b, e1a, e1b, src, dst, w, out_a, out_b, accum, rows,
             sbA, dbA, wbA, sbB, dbB, wbB, *sems) = refs
        else:
            (tab_a, tab_b, src, dst, w, out_a, out_b, accum, rows,
             sbA, dbA, wbA, sbB, dbB, wbB, *sems) = refs
        gsems = sems[:RING]
        ssems = sems[RING:2 * RING]
        isA, isB = sems[2 * RING], sems[2 * RING + 1]
        cid = lax.axis_index("c")
        sid = lax.axis_index("s")
        row0 = sid * rpt

        def zb(i, _):
            rows[i, pl.ds(0, 16)] = jnp.zeros((16,), jnp.float32)
            rows[i, pl.ds(16, 16)] = jnp.zeros((16,), jnp.float32)
            return 0
        lax.fori_loop(0, ZR, zb, 0)

        def zchunk(k, _):
            c = sid + k * NTILE

            @pl.when(c < wchunks)
            def _():
                pltpu.sync_copy(rows.at[pl.ds(0, ZR)],
                                accum.at[pl.ds(c * ZR, ZR)])
            return 0
        lax.fori_loop(0, (wchunks + NTILE - 1) // NTILE, zchunk, 0)
        plsc.subcore_barrier()

        @pl.when(cid == 0)
        def _():
            _edge_phase(tab_a, accum, src, dst, w, rows, sbA, dbA, wbA,
                        sbB, dbB, wbB, gsems, ssems, isA, isB, row0, niter)

        @pl.when(cid == 1)
        def _():
            _edge_phase(tab_b, accum, src, dst, w, rows, sbA, dbA, wbA,
                        sbB, dbB, wbB, gsems, ssems, isA, isB, row0, niter)
        plsc.subcore_barrier()

        if final:
            @pl.when(cid == 0)
            def _():
                _writeout_mean(accum, e1a, tab_a, out_a, rows, sid, wchunks)

            @pl.when(cid == 1)
            def _():
                _writeout_mean(accum, e1b, tab_b, out_b, rows, sid, wchunks)
        else:
            @pl.when(cid == 0)
            def _():
                _writeout(accum, out_a, sid, wchunks)

            @pl.when(cid == 1)
            def _():
                _writeout(accum, out_b, sid, wchunks)

    return pl.kernel(body, mesh=mesh, out_type=out_type,
                     scratch_types=scratch,
                     compiler_params=pltpu.CompilerParams(
                         use_tc_tiling_on_sc=False))


@jax.jit
def kernel(user_emb, item_emb, edge_index, edge_weight):
    n_user = user_emb.shape[0]
    n = n_user + item_emb.shape[0]
    emb = user_emb.shape[1]
    h = emb // 2
    e = edge_weight.shape[0]

    ego = jnp.concatenate([user_emb, item_emb], axis=0)
    ego_a = ego[:, :h]
    ego_b = ego[:, h:]

    quant = NTILE * UNROLL * LANES
    e_pad = -(-e // quant) * quant
    slack = GR * LANES
    src = jnp.pad(edge_index[1].astype(jnp.int32), (0, e_pad + slack - e))
    dst = jnp.pad(edge_index[0].astype(jnp.int32), (0, e_pad + slack - e))
    w = jnp.pad(edge_weight, (0, e_pad + slack - e))
    rows_total = e_pad // LANES
    src = src.reshape(rows_total + GR, LANES)
    dst = dst.reshape(rows_total + GR, LANES)
    w = w.reshape(rows_total + GR, LANES)

    layer = _make_layer(n, h, rows_total, final=False)
    layer_fin = _make_layer(n, h, rows_total, final=True)

    e1a, e1b = layer(ego_a, ego_b, src, dst, w)
    e2a, e2b = layer(e1a, e1b, src, dst, w)
    fa, fb = layer_fin(e2a, e2b, e1a, e1b, src, dst, w)

    fin = jnp.concatenate([fa, fb], axis=1)
    return fin[:n_user], fin[n_user:]
